# baseline (speedup 1.0000x reference)
import functools

import jax
import jax.numpy as jnp
from jax import lax
from jax.experimental import pallas as pl
from jax.experimental.pallas import tpu as pltpu
from jax.experimental.pallas import tpu_sc as plsc

VOCAB = 10000
DIM = 128
N_OUT = 128
B = 4096
L = 50

NC = 2
NS = 16
NW = NC * NS
TOKENS = B * L
BATCHES_PER_W = B // NW
CB = 4
NCH = BATCHES_PER_W // CB
NBUF = 4

ROWS_BLK = 1000


def _fold_body(emb_ref, w_ref, b_ref, out_ref):
    out_ref[...] = lax.dot_general(
        emb_ref[...], w_ref[...],
        dimension_numbers=(((1,), (1,)), ((), ())),
        preferred_element_type=jnp.float32,
    ) + b_ref[...]


def _fold_table(emb, W, b2):
    return pl.pallas_call(
        _fold_body,
        grid=(VOCAB // ROWS_BLK,),
        in_specs=[
            pl.BlockSpec((ROWS_BLK, DIM), lambda i: (i, 0)),
            pl.BlockSpec((N_OUT, DIM), lambda i: (0, 0)),
            pl.BlockSpec((1, N_OUT), lambda i: (0, 0)),
        ],
        out_specs=pl.BlockSpec((ROWS_BLK, N_OUT), lambda i: (i, 0)),
        out_shape=jax.ShapeDtypeStruct((VOCAB, N_OUT), jnp.float32),
    )(emb, W, b2)


def _sc_body(t_hbm, x_hbm, out_hbm, idx_v, rows_v, gsems, osems):
    wid = lax.axis_index("s") * NC + lax.axis_index("c")
    base = wid * BATCHES_PER_W

    pltpu.sync_copy(x_hbm.at[wid], idx_v)

    def start_gathers(c, buf):
        for j in range(CB):
            pltpu.async_copy(
                t_hbm.at[idx_v.at[c * CB + j]], rows_v.at[buf].at[j],
                gsems.at[buf])

    def wait_gathers(c, buf):
        for j in range(CB):
            pltpu.make_async_copy(
                t_hbm.at[idx_v.at[c * CB + j]], rows_v.at[buf].at[j],
                gsems.at[buf]).wait()

    def start_out(c, buf):
        pltpu.async_copy(
            rows_v.at[buf], out_hbm.at[pl.ds(base + c * CB, CB)],
            osems.at[buf])

    def wait_out(c, buf):
        pltpu.make_async_copy(
            rows_v.at[buf], out_hbm.at[pl.ds(base + c * CB, CB)],
            osems.at[buf]).wait()

    for b in range(NBUF):
        start_gathers(b, b)

    @pl.loop(0, NCH - NBUF, step=NBUF)
    def _group(g):
        for b in range(NBUF):
            wait_gathers(g + b, b)
            start_out(g + b, b)
        for b in range(NBUF):
            wait_out(g + b, b)
            start_gathers(g + b + NBUF, b)

    for b in range(NBUF):
        c = NCH - NBUF + b
        wait_gathers(c, b)
        start_out(c, b)
    for b in range(NBUF):
        wait_out(NCH - NBUF + b, b)


def _sc_gather(T, x3):
    mesh = plsc.VectorSubcoreMesh(
        core_axis_name="c", subcore_axis_name="s", num_cores=NC,
        num_subcores=NS)
    run = pl.kernel(
        _sc_body,
        out_type=jax.ShapeDtypeStruct((B, L, N_OUT), jnp.float32),
        mesh=mesh,
        scratch_types=[
            pltpu.VMEM((BATCHES_PER_W, L), jnp.int32),
            pltpu.VMEM((NBUF, CB, L, N_OUT), jnp.float32),
            pltpu.SemaphoreType.DMA((NBUF,)),
            pltpu.SemaphoreType.DMA((NBUF,)),
        ],
    )
    return run(T, x3)


@jax.jit
def kernel(x, emb, W, b):
    T = _fold_table(emb, W, b.reshape(1, N_OUT))
    x3 = x.astype(jnp.int32).reshape(NW, BATCHES_PER_W, L)
    return _sc_gather(T, x3)
